# baseline (speedup 1.0000x reference)
import functools

import jax
import jax.numpy as jnp
import numpy as np
from jax import lax
from jax.experimental import pallas as pl
from jax.experimental.pallas import tpu as pltpu
from jax.experimental.pallas import tpu_sc as plsc

_FIELD_DIMS = [100000] * 26
_NF = len(_FIELD_DIMS)
_B = 16384
_V = 2600000
_L = 16
_NC, _NS = 2, 16
_NW = _NC * _NS
_BPW = _B // _NW
_GPW = _BPW * _NF
_CHUNKS = _BPW // _L

_OFFSETS = np.concatenate(([0], np.cumsum(np.array(_FIELD_DIMS))[:-1])).astype(np.int32)

_mesh = plsc.VectorSubcoreMesh(core_axis_name="c", subcore_axis_name="s")


_FBLK = 163840
_FGRID = (_V + _FBLK - 1) // _FBLK


def _flatten_body(t_ref, o_ref):
    o_ref[...] = t_ref[0, :]


_tc_flatten_call = pl.pallas_call(
    _flatten_body,
    grid=(_FGRID,),
    in_specs=[pl.BlockSpec((1, _FBLK), lambda i: (0, i))],
    out_specs=pl.BlockSpec((_FBLK,), lambda i: (i,)),
    out_shape=jax.ShapeDtypeStruct((_V,), jnp.float32),
)


def _tc_flatten(table):
    return _tc_flatten_call(lax.transpose(table, (1, 0)))


@functools.partial(
    pl.kernel,
    out_type=jax.ShapeDtypeStruct((_B,), jnp.float32),
    mesh=_mesh,
    scratch_types=[
        pltpu.VMEM((_GPW,), jnp.int32),
        pltpu.VMEM((_GPW,), jnp.float32),
        pltpu.VMEM((_BPW,), jnp.float32),
        pltpu.SemaphoreType.DMA,
    ],
    compiler_params=pltpu.CompilerParams(needs_layout_passes=False),
)
def _sc_lookup(idx_hbm, table_hbm, out_hbm, idx_v, rows_v, out_v, sem):
    wid = lax.axis_index("s") * _NC + lax.axis_index("c")
    gbase = wid * _GPW
    obase = wid * _BPW

    pltpu.sync_copy(idx_hbm.at[pl.ds(gbase, _GPW)], idx_v)
    pltpu.async_copy(table_hbm.at[idx_v], rows_v, sem).wait()

    lanes = lax.iota(jnp.int32, _L) * _NF

    def chunk_body(c, _):
        base = c * (_L * _NF)
        acc = jnp.zeros((_L,), jnp.float32)
        for f in range(_NF):
            acc = acc + plsc.load_gather(rows_v, [lanes + (base + f)])
        out_v[pl.ds(c * _L, _L)] = acc
        return _

    lax.fori_loop(0, _CHUNKS, chunk_body, None)
    pltpu.sync_copy(out_v, out_hbm.at[pl.ds(obase, _BPW)])


def kernel(x, table, bias):
    offsets = jnp.asarray(_OFFSETS)
    idx = (x + offsets[None, :]).reshape(-1)
    wx = _sc_lookup(idx, _tc_flatten(table))
    return wx[:, None] + bias[None, :]
